# baseline (speedup 1.0000x reference)
import functools

import jax
import jax.numpy as jnp
from jax import lax
from jax.experimental import pallas as pl
from jax.experimental.pallas import tpu as pltpu
from jax.experimental.pallas import tpu_sc as plsc

NUM_NUMERICAL = 13
NUM_EMBED = 26
VOCAB = 100000
D = 64
BATCH = 16384
OUT_W = NUM_NUMERICAL + NUM_EMBED * D

HALF = NUM_EMBED // 2

NC = 2
NS = 16
NW = NC * NS

IDX_HALF = BATCH * HALF
IDX_PER_W = IDX_HALF // NW
CHUNK = 128
GATHERS_PER_STEP = 4
STEP = CHUNK * GATHERS_PER_STEP
STEPS = IDX_PER_W // STEP

_VB = 4096
_VBLKS = (VOCAB + _VB - 1) // _VB
_QB = _VB // 4
_RPT = _VBLKS * _QB


def _pack_bf16(a, b):
    ua = lax.bitcast_convert_type(a, jnp.uint32)
    ub = lax.bitcast_convert_type(b, jnp.uint32)
    ra = (ua + 0x7FFF + ((ua >> 16) & 1)) >> 16
    rb = (ub + 0x7FFF + ((ub >> 16) & 1)) & jnp.uint32(0xFFFF0000)
    return lax.bitcast_convert_type(ra | rb, jnp.float32)


def _relayout_body(t_ref, o_ref):
    t = t_ref[0].T
    o_ref[0] = jnp.concatenate(
        [_pack_bf16(t[0:_QB], t[_QB:2 * _QB]),
         _pack_bf16(t[2 * _QB:3 * _QB], t[3 * _QB:4 * _QB])], axis=1)


def _make_relayout(j0):
    return pl.pallas_call(
        _relayout_body,
        out_shape=jax.ShapeDtypeStruct((HALF, _RPT, 2 * D), jnp.float32),
        grid=(HALF, _VBLKS),
        in_specs=[pl.BlockSpec((1, D, _VB), lambda j, k: (j0 + j, 0, k))],
        out_specs=pl.BlockSpec((1, _QB, 2 * D), lambda j, k: (j, k, 0)),
    )


_relayout0 = _make_relayout(0)
_relayout1 = _make_relayout(HALF)

_mesh = plsc.VectorSubcoreMesh(core_axis_name="c", subcore_axis_name="s")


@functools.partial(
    pl.kernel,
    out_type=jax.ShapeDtypeStruct((IDX_HALF, 2 * D), jnp.float32),
    mesh=_mesh,
    scratch_types=[
        pltpu.VMEM((IDX_PER_W,), jnp.int32),
        pltpu.VMEM((STEP, 2 * D), jnp.float32),
        pltpu.SemaphoreType.DMA,
    ],
)
def _sc_gather(tables_hbm, idx_hbm, out_hbm, idx_v, buf_v, sem):
    wid = lax.axis_index("s") * NC + lax.axis_index("c")
    base = wid * IDX_PER_W
    pltpu.sync_copy(idx_hbm.at[pl.ds(base, IDX_PER_W)], idx_v)

    @pl.loop(0, STEPS)
    def _(step):
        off = step * STEP
        copies = []
        for g in range(GATHERS_PER_STEP):
            copies.append(
                pltpu.async_copy(
                    tables_hbm.at[idx_v.at[pl.ds(off + g * CHUNK, CHUNK)]],
                    buf_v.at[pl.ds(g * CHUNK, CHUNK)],
                    sem,
                )
            )
        for c in copies:
            c.wait()
        pltpu.sync_copy(buf_v, out_hbm.at[pl.ds(base + off, STEP)])


_RB = 512


def _concat_body(x_ref, emb0_ref, emb1_ref, o_ref):
    o_ref[:, 0:NUM_NUMERICAL] = x_ref[:, 0:NUM_NUMERICAL]
    for j in range(NUM_EMBED):
        e_ref = emb0_ref if j < HALF else emb1_ref
        jj = j % HALF
        i = x_ref[:, NUM_NUMERICAL + j].astype(jnp.int32)
        qd = (i % _VB) // _QB
        eh = jnp.where((qd < 2)[:, None],
                       e_ref[jj, :, 0:D], e_ref[jj, :, D:2 * D])
        u = lax.bitcast_convert_type(eh, jnp.uint32)
        bits = jnp.where((qd % 2 == 0)[:, None],
                         u << 16, u & jnp.uint32(0xFFFF0000))
        e = lax.bitcast_convert_type(bits, jnp.float32)
        col = NUM_NUMERICAL + j * D
        o_ref[:, col:col + D] = e


_concat = pl.pallas_call(
    _concat_body,
    out_shape=jax.ShapeDtypeStruct((BATCH, OUT_W), jnp.float32),
    grid=(BATCH // _RB,),
    in_specs=[
        pl.BlockSpec((_RB, NUM_NUMERICAL + NUM_EMBED), lambda i: (i, 0)),
        pl.BlockSpec((HALF, _RB, 2 * D), lambda i: (0, i, 0)),
        pl.BlockSpec((HALF, _RB, 2 * D), lambda i: (0, i, 0)),
    ],
    out_specs=pl.BlockSpec((_RB, OUT_W), lambda i: (i, 0)),
)


def kernel(x, tables):
    tables_cm = jnp.swapaxes(tables, 1, 2)
    i = x[:, NUM_NUMERICAL:].astype(jnp.int32).T
    row = ((i // _VB) * _QB + i % _QB
           + (jnp.arange(NUM_EMBED, dtype=jnp.int32) % HALF * _RPT)[:, None])
    trows0 = _relayout0(tables_cm).reshape(HALF * _RPT, 2 * D)
    emb0 = _sc_gather(trows0, row[:HALF].reshape(-1))
    trows1 = _relayout1(tables_cm).reshape(HALF * _RPT, 2 * D)
    emb1 = _sc_gather(trows1, row[HALF:].reshape(-1))
    return _concat(x, emb0.reshape(HALF, BATCH, 2 * D),
                   emb1.reshape(HALF, BATCH, 2 * D))


# speedup vs baseline: 1.0353x; 1.0353x over previous
import dataclasses
import functools

import jax
import jax.numpy as jnp
from jax import lax
from jax.experimental import pallas as pl
from jax.experimental.pallas import tpu as pltpu
from jax.experimental.pallas import tpu_sc as plsc

NUM_NUMERICAL = 13
NUM_EMBED = 26
VOCAB = 100000
D = 64
BATCH = 16384
OUT_W = NUM_NUMERICAL + NUM_EMBED * D

HALF = NUM_EMBED // 2

NC = 2
NS = 16
NW = NC * NS

IDX_HALF = BATCH * HALF
IDX_PER_W = IDX_HALF // NW
CHUNK = 128
GATHERS_PER_STEP = 2
STEP = CHUNK * GATHERS_PER_STEP
STEPS = IDX_PER_W // STEP

_VB = 4096
_VBLKS = (VOCAB + _VB - 1) // _VB
_QB = _VB // 4
_RPT = _VBLKS * _QB


def _relayout_body(t_ref, o_ref):
    u = lax.bitcast_convert_type(t_ref[0], jnp.uint32)
    r = (u + 0x7FFF + ((u >> 16) & 1)) >> 16
    w = r[0:D // 2] | (r[D // 2:D] << 16)
    t = lax.bitcast_convert_type(w, jnp.float32).T
    o_ref[0] = jnp.concatenate(
        [t[q * _QB:(q + 1) * _QB] for q in range(4)], axis=1)


def _make_relayout(j0):
    return pl.pallas_call(
        _relayout_body,
        out_shape=jax.ShapeDtypeStruct((HALF, _RPT, 2 * D), jnp.float32),
        grid=(HALF, _VBLKS),
        in_specs=[pl.BlockSpec((1, D, _VB), lambda j, k: (j0 + j, 0, k))],
        out_specs=pl.BlockSpec((1, _QB, 2 * D), lambda j, k: (j, k, 0)),
    )


_relayout0 = _make_relayout(0)
_relayout1 = _make_relayout(HALF)

_mesh = plsc.VectorSubcoreMesh(core_axis_name="c", subcore_axis_name="s")


_W32 = D // 2


@functools.partial(
    pl.kernel,
    out_type=jax.ShapeDtypeStruct((IDX_HALF, _W32), jnp.float32),
    mesh=_mesh,
    scratch_types=[
        pltpu.VMEM((IDX_PER_W,), jnp.int32),
        pltpu.VMEM((IDX_PER_W,), jnp.int32),
        pltpu.VMEM((STEP, 2 * D), jnp.float32),
        pltpu.VMEM((STEP, _W32), jnp.float32),
        pltpu.SemaphoreType.DMA,
    ],
    compiler_params=dataclasses.replace(
        pltpu.CompilerParams(), needs_layout_passes=False)
    if "needs_layout_passes" in pltpu.CompilerParams.__dataclass_fields__
    else None,
)
def _sc_gather(tables_hbm, idx_hbm, qd_hbm, out_hbm,
               idx_v, qd_v, buf_v, cbuf_v, sem):
    wid = lax.axis_index("s") * NC + lax.axis_index("c")
    base = wid * IDX_PER_W
    pltpu.sync_copy(idx_hbm.at[pl.ds(base, IDX_PER_W)], idx_v)
    pltpu.sync_copy(qd_hbm.at[pl.ds(base, IDX_PER_W)], qd_v)

    @pl.loop(0, STEPS)
    def _(step):
        off = step * STEP
        copies = []
        for g in range(GATHERS_PER_STEP):
            copies.append(
                pltpu.async_copy(
                    tables_hbm.at[idx_v.at[pl.ds(off + g * CHUNK, CHUNK)]],
                    buf_v.at[pl.ds(g * CHUNK, CHUNK)],
                    sem,
                )
            )
        for c in copies:
            c.wait()

        @pl.loop(0, STEP, step=16)
        def _(r0):
            rows = lax.iota(jnp.int32, 16) + r0
            cols0 = qd_v[pl.ds(off + r0, 16)] * _W32
            for c in range(_W32):
                val = plsc.load_gather(buf_v, [rows, cols0 + c])
                plsc.store_scatter(
                    cbuf_v, [rows, jnp.full((16,), c, jnp.int32)], val)

        pltpu.sync_copy(cbuf_v, out_hbm.at[pl.ds(base + off, STEP)])


_RB = 512


def _concat_body(x_ref, emb0_ref, emb1_ref, o_ref):
    o_ref[0:NUM_NUMERICAL, :] = x_ref[:, 0:NUM_NUMERICAL].T
    for j in range(NUM_EMBED):
        e_ref = emb0_ref if j < HALF else emb1_ref
        jj = j % HALF
        u = lax.bitcast_convert_type(e_ref[jj], jnp.uint32)
        e = jnp.concatenate(
            [lax.bitcast_convert_type(u << 16, jnp.float32),
             lax.bitcast_convert_type(u & jnp.uint32(0xFFFF0000),
                                      jnp.float32)], axis=1)
        col = NUM_NUMERICAL + j * D
        o_ref[col:col + D, :] = e.T


_concat = pl.pallas_call(
    _concat_body,
    out_shape=jax.ShapeDtypeStruct((OUT_W, BATCH), jnp.float32),
    grid=(BATCH // _RB,),
    in_specs=[
        pl.BlockSpec((_RB, NUM_NUMERICAL + NUM_EMBED), lambda i: (i, 0)),
        pl.BlockSpec((HALF, _RB, _W32), lambda i: (0, i, 0)),
        pl.BlockSpec((HALF, _RB, _W32), lambda i: (0, i, 0)),
    ],
    out_specs=pl.BlockSpec((OUT_W, _RB), lambda i: (0, i)),
)


def kernel(x, tables):
    tables_cm = jnp.swapaxes(tables, 1, 2)
    i = x[:, NUM_NUMERICAL:].astype(jnp.int32).T
    row = ((i // _VB) * _QB + i % _QB
           + (jnp.arange(NUM_EMBED, dtype=jnp.int32) % HALF * _RPT)[:, None])
    qd = (i % _VB) // _QB
    trows0 = _relayout0(tables_cm).reshape(HALF * _RPT, 2 * D)
    emb0 = _sc_gather(trows0, row[:HALF].reshape(-1), qd[:HALF].reshape(-1))
    trows1 = _relayout1(tables_cm).reshape(HALF * _RPT, 2 * D)
    emb1 = _sc_gather(trows1, row[HALF:].reshape(-1), qd[HALF:].reshape(-1))
    out_t = _concat(x, emb0.reshape(HALF, BATCH, _W32),
                    emb1.reshape(HALF, BATCH, _W32))
    return out_t.T


# speedup vs baseline: 1.0774x; 1.0407x over previous
import functools

import jax
import jax.numpy as jnp
from jax import lax
from jax.experimental import pallas as pl
from jax.experimental.pallas import tpu as pltpu
from jax.experimental.pallas import tpu_sc as plsc

NUM_NUMERICAL = 13
NUM_EMBED = 26
VOCAB = 100000
D = 64
BATCH = 16384
OUT_W = NUM_NUMERICAL + NUM_EMBED * D

HALF = NUM_EMBED // 2

NC = 2
NS = 16
NW = NC * NS

IDX_HALF = BATCH * HALF
IDX_PER_W = IDX_HALF // NW
CHUNK = 128
GATHERS_PER_STEP = 4
STEP = CHUNK * GATHERS_PER_STEP
STEPS = IDX_PER_W // STEP

_VB = 4096
_VBLKS = (VOCAB + _VB - 1) // _VB
_QB = _VB // 4
_RPT = _VBLKS * _QB


def _pack_bf16(a, b):
    ua = lax.bitcast_convert_type(a, jnp.uint32)
    ub = lax.bitcast_convert_type(b, jnp.uint32)
    ra = (ua + 0x7FFF + ((ua >> 16) & 1)) >> 16
    rb = (ub + 0x7FFF + ((ub >> 16) & 1)) & jnp.uint32(0xFFFF0000)
    return lax.bitcast_convert_type(ra | rb, jnp.float32)


def _relayout_body(t_ref, o_ref):
    t = t_ref[0].T
    o_ref[0] = jnp.concatenate(
        [_pack_bf16(t[0:_QB], t[_QB:2 * _QB]),
         _pack_bf16(t[2 * _QB:3 * _QB], t[3 * _QB:4 * _QB])], axis=1)


def _make_relayout(j0):
    return pl.pallas_call(
        _relayout_body,
        out_shape=jax.ShapeDtypeStruct((HALF, _RPT, 2 * D), jnp.float32),
        grid=(HALF, _VBLKS),
        in_specs=[pl.BlockSpec((1, D, _VB), lambda j, k: (j0 + j, 0, k))],
        out_specs=pl.BlockSpec((1, _QB, 2 * D), lambda j, k: (j, k, 0)),
    )


_relayout0 = _make_relayout(0)
_relayout1 = _make_relayout(HALF)

_mesh = plsc.VectorSubcoreMesh(core_axis_name="c", subcore_axis_name="s")


@functools.partial(
    pl.kernel,
    out_type=jax.ShapeDtypeStruct((IDX_HALF, 2 * D), jnp.float32),
    mesh=_mesh,
    scratch_types=[
        pltpu.VMEM((IDX_PER_W,), jnp.int32),
        pltpu.VMEM((STEP, 2 * D), jnp.float32),
        pltpu.SemaphoreType.DMA,
    ],
)
def _sc_gather(tables_hbm, idx_hbm, out_hbm, idx_v, buf_v, sem):
    wid = lax.axis_index("s") * NC + lax.axis_index("c")
    base = wid * IDX_PER_W
    pltpu.sync_copy(idx_hbm.at[pl.ds(base, IDX_PER_W)], idx_v)

    @pl.loop(0, STEPS)
    def _(step):
        off = step * STEP
        copies = []
        for g in range(GATHERS_PER_STEP):
            copies.append(
                pltpu.async_copy(
                    tables_hbm.at[idx_v.at[pl.ds(off + g * CHUNK, CHUNK)]],
                    buf_v.at[pl.ds(g * CHUNK, CHUNK)],
                    sem,
                )
            )
        for c in copies:
            c.wait()
        pltpu.sync_copy(buf_v, out_hbm.at[pl.ds(base + off, STEP)])


_RB = 512


def _concat_body(x_ref, emb0_ref, emb1_ref, o_ref):
    o_ref[0:NUM_NUMERICAL, :] = x_ref[:, 0:NUM_NUMERICAL].T
    for j in range(NUM_EMBED):
        e_ref = emb0_ref if j < HALF else emb1_ref
        jj = j % HALF
        i = x_ref[:, NUM_NUMERICAL + j].astype(jnp.int32)
        qd = (i % _VB) // _QB
        eh = jnp.where((qd < 2)[:, None],
                       e_ref[jj, :, 0:D], e_ref[jj, :, D:2 * D])
        u = lax.bitcast_convert_type(eh, jnp.uint32)
        bits = jnp.where((qd % 2 == 0)[:, None],
                         u << 16, u & jnp.uint32(0xFFFF0000))
        e = lax.bitcast_convert_type(bits, jnp.float32)
        col = NUM_NUMERICAL + j * D
        o_ref[col:col + D, :] = e.T


_concat = pl.pallas_call(
    _concat_body,
    out_shape=jax.ShapeDtypeStruct((OUT_W, BATCH), jnp.float32),
    grid=(BATCH // _RB,),
    in_specs=[
        pl.BlockSpec((_RB, NUM_NUMERICAL + NUM_EMBED), lambda i: (i, 0)),
        pl.BlockSpec((HALF, _RB, 2 * D), lambda i: (0, i, 0)),
        pl.BlockSpec((HALF, _RB, 2 * D), lambda i: (0, i, 0)),
    ],
    out_specs=pl.BlockSpec((OUT_W, _RB), lambda i: (0, i)),
)


def kernel(x, tables):
    tables_cm = jnp.swapaxes(tables, 1, 2)
    i = x[:, NUM_NUMERICAL:].astype(jnp.int32).T
    row = ((i // _VB) * _QB + i % _QB
           + (jnp.arange(NUM_EMBED, dtype=jnp.int32) % HALF * _RPT)[:, None])
    trows0 = _relayout0(tables_cm).reshape(HALF * _RPT, 2 * D)
    emb0 = _sc_gather(trows0, row[:HALF].reshape(-1))
    trows1 = _relayout1(tables_cm).reshape(HALF * _RPT, 2 * D)
    emb1 = _sc_gather(trows1, row[HALF:].reshape(-1))
    out_t = _concat(x, emb0.reshape(HALF, BATCH, 2 * D),
                    emb1.reshape(HALF, BATCH, 2 * D))
    return out_t.T
